# baseline (speedup 1.0000x reference)
import functools

import jax
import jax.numpy as jnp
from jax import lax
from jax.experimental import pallas as pl
from jax.experimental.pallas import tpu as pltpu
from jax.experimental.pallas import tpu_sc as plsc

_N = 10000
_D = 128
_E = 320000
_NC = 2
_NS = 16
_NW = _NC * _NS
_EPT = _E // _NW
_K = 80
_NCH = _EPT // _K
_RPT = _N // _NS

@functools.lru_cache(maxsize=None)
def _make_sc_agg():
    mesh = plsc.VectorSubcoreMesh(core_axis_name="c", subcore_axis_name="s",
                                  num_cores=_NC, num_subcores=_NS)
    return functools.partial(
        pl.kernel,
        out_type=jax.ShapeDtypeStruct((_NC, _N, _D), jnp.float32),
        mesh=mesh,
        scratch_types=[
        pltpu.VMEM_SHARED((_N, _D), jnp.float32),
        pltpu.VMEM((_EPT,), jnp.int32),
        [pltpu.VMEM((_K,), jnp.int32) for _ in range(4)],
        [pltpu.VMEM((_K, _D), jnp.float32) for _ in range(2)],
        pltpu.SemaphoreType.DMA,
        [pltpu.SemaphoreType.DMA for _ in range(4)],
        [pltpu.SemaphoreType.DMA for _ in range(2)],
        [pltpu.SemaphoreType.DMA for _ in range(2)],
    ],
    )(_sc_agg_body)


def _sc_agg_body(x_hbm, src_hbm, dst_hbm, out_hbm,
                 agg_sh, sb, dbs, rbs, semst, semds, semgs, semss):
    c = lax.axis_index("c")
    s = lax.axis_index("s")
    wid = s * _NC + c
    base = wid * _EPT
    rb0 = rbs[0]

    pltpu.async_copy(src_hbm.at[pl.ds(base, _EPT)], sb, semst)

    zero16 = jnp.zeros((16,), jnp.float32)

    def _zrow(i, _):
        for j in range(_D // 16):
            rb0[i, pl.ds(j * 16, 16)] = zero16
        return 0

    lax.fori_loop(0, _K, _zrow, 0)
    r0 = s * _RPT
    for q in range(_RPT // _K):
        pltpu.sync_copy(rb0, agg_sh.at[pl.ds(r0 + q * _K, _K)])
    rem = _RPT % _K
    if rem:
        pltpu.sync_copy(rb0.at[pl.ds(0, rem)],
                        agg_sh.at[pl.ds(r0 + (_RPT // _K) * _K, rem)])
    pltpu.make_async_copy(src_hbm.at[pl.ds(base, _EPT)], sb, semst).wait()
    plsc.subcore_barrier()

    def _dst_start(cc, b):
        pltpu.async_copy(dst_hbm.at[pl.ds(base + cc * _K, _K)], dbs[b],
                         semds[b])

    def _dst_wait(cc, b):
        pltpu.make_async_copy(dst_hbm.at[pl.ds(base + cc * _K, _K)], dbs[b],
                              semds[b]).wait()

    def _gather_start(cc, b):
        pltpu.async_copy(x_hbm.at[sb.at[pl.ds(cc * _K, _K)]], rbs[b],
                         semgs[b])

    def _gather_wait(cc, b):
        pltpu.make_async_copy(x_hbm.at[sb.at[pl.ds(cc * _K, _K)]], rbs[b],
                              semgs[b]).wait()

    def _scatter_start(b2, b4):
        pltpu.async_copy(rbs[b2], agg_sh.at[dbs[b4]], semss[b2], add=True)

    def _scatter_wait(b2, b4):
        pltpu.make_async_copy(rbs[b2], agg_sh.at[dbs[b4]], semss[b2]).wait()

    def _slot(cc, b2, b4, first=False):
        _gather_wait(cc, b2)
        _dst_wait(cc, b4)
        _scatter_start(b2, b4)
        if not first:
            _scatter_wait(1 - b2, (b4 + 3) % 4)

        @pl.when(cc + 1 < _NCH)
        def _():
            _gather_start(cc + 1, 1 - b2)

        @pl.when(cc + 3 < _NCH)
        def _():
            _dst_start(cc + 3, (b4 + 3) % 4)

    _dst_start(0, 0)
    _dst_start(1, 1)
    _dst_start(2, 2)
    _gather_start(0, 0)
    _slot(0, 0, 0, first=True)
    _slot(1, 1, 1)
    _slot(2, 0, 2)
    _slot(3, 1, 3)

    def _body(j, _):
        c0 = 4 * j
        for k in range(4):
            _slot(c0 + k, k % 2, k)
        return 0

    lax.fori_loop(1, (_NCH - 1) // 4, _body, 0)
    _slot(_NCH - 1, 0, 0)
    _scatter_wait(0, 0)

    plsc.subcore_barrier()
    rw = _N // _NS // 8 * 8
    r0w = s * rw
    pltpu.sync_copy(agg_sh.at[pl.ds(r0w, rw)], out_hbm.at[c, pl.ds(r0w, rw)])

    @pl.when(s == _NS - 1)
    def _():
        pltpu.sync_copy(agg_sh.at[pl.ds(_NS * rw, _N - _NS * rw)],
                        out_hbm.at[c, pl.ds(_NS * rw, _N - _NS * rw)])


_BLK = 1000
_GRID = _N // _BLK


def _mm_stats_body(x_ref, p_ref, wa_ref, ba_ref, wb_ref, bb_ref, eps_ref,
                   h2_ref, st_ref):
    i = pl.program_id(0)
    t = (1.0 + eps_ref[0, 0]) * x_ref[...] + p_ref[0] + p_ref[1]
    h1 = jnp.maximum(
        jnp.dot(t.astype(jnp.bfloat16), wa_ref[...].astype(jnp.bfloat16),
                preferred_element_type=jnp.float32)
        + ba_ref[...], 0.0)
    h2 = jnp.dot(h1.astype(jnp.bfloat16), wb_ref[...].astype(jnp.bfloat16),
                 preferred_element_type=jnp.float32) + bb_ref[...]
    h2_ref[...] = h2
    st = jnp.concatenate(
        [jnp.sum(h2, axis=0, keepdims=True),
         jnp.sum(h2 * h2, axis=0, keepdims=True)], axis=0)

    @pl.when(i == 0)
    def _():
        st_ref[...] = st

    @pl.when(i > 0)
    def _():
        st_ref[...] = st_ref[...] + st


def _mm_stats(x, parts, wa, ba, wb, bb, eps):
    return pl.pallas_call(
        _mm_stats_body,
        grid=(_GRID,),
        in_specs=[
            pl.BlockSpec((_BLK, _D), lambda i: (i, 0)),
            pl.BlockSpec((_NC, _BLK, _D), lambda i: (0, i, 0)),
            pl.BlockSpec((_D, _D), lambda i: (0, 0)),
            pl.BlockSpec((1, _D), lambda i: (0, 0)),
            pl.BlockSpec((_D, _D), lambda i: (0, 0)),
            pl.BlockSpec((1, _D), lambda i: (0, 0)),
            pl.BlockSpec((1, 1), lambda i: (0, 0)),
        ],
        out_specs=[
            pl.BlockSpec((_BLK, _D), lambda i: (i, 0)),
            pl.BlockSpec((2, _D), lambda i: (0, 0)),
        ],
        out_shape=[
            jax.ShapeDtypeStruct((_N, _D), jnp.float32),
            jax.ShapeDtypeStruct((2, _D), jnp.float32),
        ],
    )(x, parts, wa, ba.reshape(1, _D), wb, bb.reshape(1, _D),
      eps.reshape(1, 1))


def _norm_body(h2_ref, st_ref, g_ref, b_ref, out_ref):
    mu = st_ref[0:1, :] * (1.0 / _N)
    var = st_ref[1:2, :] * (1.0 / _N) - mu * mu
    scale = lax.rsqrt(var + 1e-5) * g_ref[...]
    out_ref[...] = jnp.maximum((h2_ref[...] - mu) * scale + b_ref[...], 0.0)


def _norm(h2, st, g, beta):
    return pl.pallas_call(
        _norm_body,
        grid=(_GRID,),
        in_specs=[
            pl.BlockSpec((_BLK, _D), lambda i: (i, 0)),
            pl.BlockSpec((2, _D), lambda i: (0, 0)),
            pl.BlockSpec((1, _D), lambda i: (0, 0)),
            pl.BlockSpec((1, _D), lambda i: (0, 0)),
        ],
        out_specs=pl.BlockSpec((_BLK, _D), lambda i: (i, 0)),
        out_shape=jax.ShapeDtypeStruct((_N, _D), jnp.float32),
    )(h2, st, g.reshape(1, _D), beta.reshape(1, _D))


def _norm_head_body(h2_ref, st_ref, g_ref, b_ref, wc1_ref, bc1_ref,
                    wc2_ref, bc2_ref, out_ref):
    mu = st_ref[0:1, :] * (1.0 / _N)
    var = st_ref[1:2, :] * (1.0 / _N) - mu * mu
    scale = lax.rsqrt(var + 1e-5) * g_ref[...]
    h = jnp.maximum((h2_ref[...] - mu) * scale + b_ref[...], 0.0)
    z = jnp.maximum(
        jnp.dot(h.astype(jnp.bfloat16), wc1_ref[...].astype(jnp.bfloat16),
                preferred_element_type=jnp.float32)
        + bc1_ref[...], 0.0)
    out_ref[...] = (
        jnp.dot(z.astype(jnp.bfloat16), wc2_ref[...].astype(jnp.bfloat16),
                preferred_element_type=jnp.float32)
        + bc2_ref[...])


def _norm_head(h2, st, g, beta, wc1, bc1, wc2, bc2):
    return pl.pallas_call(
        _norm_head_body,
        grid=(_GRID,),
        in_specs=[
            pl.BlockSpec((_BLK, _D), lambda i: (i, 0)),
            pl.BlockSpec((2, _D), lambda i: (0, 0)),
            pl.BlockSpec((1, _D), lambda i: (0, 0)),
            pl.BlockSpec((1, _D), lambda i: (0, 0)),
            pl.BlockSpec((_D, _D // 2), lambda i: (0, 0)),
            pl.BlockSpec((1, _D // 2), lambda i: (0, 0)),
            pl.BlockSpec((_D // 2, 2), lambda i: (0, 0)),
            pl.BlockSpec((1, 2), lambda i: (0, 0)),
        ],
        out_specs=pl.BlockSpec((_BLK, 2), lambda i: (i, 0)),
        out_shape=jax.ShapeDtypeStruct((_N, 2), jnp.float32),
    )(h2, st, g.reshape(1, _D), beta.reshape(1, _D), wc1,
      bc1.reshape(1, _D // 2), wc2, bc2.reshape(1, 2))


def kernel(x, edge_index, W1a, b1a, W1b, b1b, eps1, g1, beta1,
           W2a, b2a, W2b, b2b, eps2, g2, beta2,
           W3a, b3a, W3b, b3b, eps3, g3, beta3,
           Wc1, bc1, Wc2, bc2):
    src = edge_index[0]
    dst = edge_index[1]
    h = x
    layers = [
        (W1a, b1a, W1b, b1b, eps1, g1, beta1),
        (W2a, b2a, W2b, b2b, eps2, g2, beta2),
        (W3a, b3a, W3b, b3b, eps3, g3, beta3),
    ]
    for li, (wa, ba, wb, bb, eps, g, beta) in enumerate(layers):
        parts = _make_sc_agg()(h, src, dst)
        h2, st = _mm_stats(h, parts, wa, ba, wb, bb, eps)
        if li < 2:
            h = _norm(h2, st, g, beta)
        else:
            out = _norm_head(h2, st, g, beta, Wc1, bc1, Wc2, bc2)
    return out


# speedup vs baseline: 1.2587x; 1.2587x over previous
import functools

import jax
import jax.numpy as jnp
from jax import lax
from jax.experimental import pallas as pl
from jax.experimental.pallas import tpu as pltpu
from jax.experimental.pallas import tpu_sc as plsc

_N = 10000
_D = 128
_E = 320000
_NC = 2
_NS = 16
_NW = _NC * _NS
_EPT = _E // _NW
_K = 80
_NCH = _EPT // _K
_RPT = _N // _NS

@functools.lru_cache(maxsize=None)
def _make_sc_agg():
    mesh = plsc.VectorSubcoreMesh(core_axis_name="c", subcore_axis_name="s",
                                  num_cores=_NC, num_subcores=_NS)
    return functools.partial(
        pl.kernel,
        out_type=jax.ShapeDtypeStruct((_NC, _N, _D), jnp.float32),
        mesh=mesh,
        scratch_types=[
        pltpu.VMEM_SHARED((_N, _D), jnp.float32),
        pltpu.VMEM((_EPT,), jnp.int32),
        pltpu.VMEM((_K,), jnp.int32),
        pltpu.VMEM((_K,), jnp.int32),
        pltpu.VMEM((_K, _D), jnp.float32),
        pltpu.VMEM((_K, _D), jnp.float32),
        pltpu.SemaphoreType.DMA,
        pltpu.SemaphoreType.DMA,
        pltpu.SemaphoreType.DMA,
        pltpu.SemaphoreType.DMA,
    ],
    )(_sc_agg_body)


def _sc_agg_body(x_hbm, src_hbm, dst_hbm, out_hbm,
                 agg_sh, sb, db0, db1, rb0, rb1, semd0, semd1, semg0, semg1):
    c = lax.axis_index("c")
    s = lax.axis_index("s")
    wid = s * _NC + c
    base = wid * _EPT

    pltpu.async_copy(src_hbm.at[pl.ds(base, _EPT)], sb, semd0)

    zero16 = jnp.zeros((16,), jnp.float32)

    def _zrow(i, _):
        for j in range(_D // 16):
            rb0[i, pl.ds(j * 16, 16)] = zero16
        return 0

    lax.fori_loop(0, _K, _zrow, 0)
    r0 = s * _RPT
    for q in range(_RPT // _K):
        pltpu.sync_copy(rb0, agg_sh.at[pl.ds(r0 + q * _K, _K)])
    rem = _RPT % _K
    if rem:
        pltpu.sync_copy(rb0.at[pl.ds(0, rem)],
                        agg_sh.at[pl.ds(r0 + (_RPT // _K) * _K, rem)])
    pltpu.make_async_copy(src_hbm.at[pl.ds(base, _EPT)], sb, semd0).wait()
    plsc.subcore_barrier()

    def _dst_start(cc, db, sem):
        pltpu.async_copy(dst_hbm.at[pl.ds(base + cc * _K, _K)], db, sem)

    def _dst_wait(cc, db, sem):
        pltpu.make_async_copy(dst_hbm.at[pl.ds(base + cc * _K, _K)], db,
                              sem).wait()

    def _gather_start(cc, rb, sem):
        pltpu.async_copy(x_hbm.at[sb.at[pl.ds(cc * _K, _K)]], rb, sem)

    def _gather_wait(cc, rb, sem):
        pltpu.make_async_copy(x_hbm.at[sb.at[pl.ds(cc * _K, _K)]], rb,
                              sem).wait()

    _dst_start(0, db0, semd0)
    _dst_start(1, db1, semd1)
    _gather_start(0, rb0, semg0)

    def _half(cc, db_cur, rb_cur, semd_cur, semg_cur, rb_nxt, semg_nxt):
        @pl.when(cc + 1 < _NCH)
        def _():
            _gather_start(cc + 1, rb_nxt, semg_nxt)

        _gather_wait(cc, rb_cur, semg_cur)
        _dst_wait(cc, db_cur, semd_cur)
        pltpu.sync_copy(rb_cur, agg_sh.at[db_cur], add=True)

        @pl.when(cc + 2 < _NCH)
        def _():
            _dst_start(cc + 2, db_cur, semd_cur)

    def _body(j, _):
        c0 = 2 * j
        _half(c0, db0, rb0, semd0, semg0, rb1, semg1)
        _half(c0 + 1, db1, rb1, semd1, semg1, rb0, semg0)
        return 0

    lax.fori_loop(0, (_NCH - 1) // 2, _body, 0)
    _gather_wait(_NCH - 1, rb0, semg0)
    _dst_wait(_NCH - 1, db0, semd0)
    pltpu.sync_copy(rb0, agg_sh.at[db0], add=True)

    plsc.subcore_barrier()
    rw = _N // _NS // 8 * 8
    r0w = s * rw
    pltpu.sync_copy(agg_sh.at[pl.ds(r0w, rw)], out_hbm.at[c, pl.ds(r0w, rw)])

    @pl.when(s == _NS - 1)
    def _():
        pltpu.sync_copy(agg_sh.at[pl.ds(_NS * rw, _N - _NS * rw)],
                        out_hbm.at[c, pl.ds(_NS * rw, _N - _NS * rw)])


_BLK = 1000
_GRID = _N // _BLK


def _mm_stats_body(x_ref, p_ref, wa_ref, ba_ref, wb_ref, bb_ref, eps_ref,
                   h2_ref, st_ref):
    i = pl.program_id(0)
    t = (1.0 + eps_ref[0, 0]) * x_ref[...] + p_ref[0] + p_ref[1]
    h1 = jnp.maximum(
        jnp.dot(t.astype(jnp.bfloat16), wa_ref[...].astype(jnp.bfloat16),
                preferred_element_type=jnp.float32)
        + ba_ref[...], 0.0)
    h2 = jnp.dot(h1.astype(jnp.bfloat16), wb_ref[...].astype(jnp.bfloat16),
                 preferred_element_type=jnp.float32) + bb_ref[...]
    h2_ref[...] = h2
    st = jnp.concatenate(
        [jnp.sum(h2, axis=0, keepdims=True),
         jnp.sum(h2 * h2, axis=0, keepdims=True)], axis=0)

    @pl.when(i == 0)
    def _():
        st_ref[...] = st

    @pl.when(i > 0)
    def _():
        st_ref[...] = st_ref[...] + st


def _mm_stats(x, parts, wa, ba, wb, bb, eps):
    return pl.pallas_call(
        _mm_stats_body,
        grid=(_GRID,),
        in_specs=[
            pl.BlockSpec((_BLK, _D), lambda i: (i, 0)),
            pl.BlockSpec((_NC, _BLK, _D), lambda i: (0, i, 0)),
            pl.BlockSpec((_D, _D), lambda i: (0, 0)),
            pl.BlockSpec((1, _D), lambda i: (0, 0)),
            pl.BlockSpec((_D, _D), lambda i: (0, 0)),
            pl.BlockSpec((1, _D), lambda i: (0, 0)),
            pl.BlockSpec((1, 1), lambda i: (0, 0)),
        ],
        out_specs=[
            pl.BlockSpec((_BLK, _D), lambda i: (i, 0)),
            pl.BlockSpec((2, _D), lambda i: (0, 0)),
        ],
        out_shape=[
            jax.ShapeDtypeStruct((_N, _D), jnp.float32),
            jax.ShapeDtypeStruct((2, _D), jnp.float32),
        ],
    )(x, parts, wa, ba.reshape(1, _D), wb, bb.reshape(1, _D),
      eps.reshape(1, 1))


def _norm_body(h2_ref, st_ref, g_ref, b_ref, out_ref):
    mu = st_ref[0:1, :] * (1.0 / _N)
    var = st_ref[1:2, :] * (1.0 / _N) - mu * mu
    scale = lax.rsqrt(var + 1e-5) * g_ref[...]
    out_ref[...] = jnp.maximum((h2_ref[...] - mu) * scale + b_ref[...], 0.0)


def _norm(h2, st, g, beta):
    return pl.pallas_call(
        _norm_body,
        grid=(_GRID,),
        in_specs=[
            pl.BlockSpec((_BLK, _D), lambda i: (i, 0)),
            pl.BlockSpec((2, _D), lambda i: (0, 0)),
            pl.BlockSpec((1, _D), lambda i: (0, 0)),
            pl.BlockSpec((1, _D), lambda i: (0, 0)),
        ],
        out_specs=pl.BlockSpec((_BLK, _D), lambda i: (i, 0)),
        out_shape=jax.ShapeDtypeStruct((_N, _D), jnp.float32),
    )(h2, st, g.reshape(1, _D), beta.reshape(1, _D))


def _norm_head_body(h2_ref, st_ref, g_ref, b_ref, wc1_ref, bc1_ref,
                    wc2_ref, bc2_ref, out_ref):
    mu = st_ref[0:1, :] * (1.0 / _N)
    var = st_ref[1:2, :] * (1.0 / _N) - mu * mu
    scale = lax.rsqrt(var + 1e-5) * g_ref[...]
    h = jnp.maximum((h2_ref[...] - mu) * scale + b_ref[...], 0.0)
    z = jnp.maximum(
        jnp.dot(h.astype(jnp.bfloat16), wc1_ref[...].astype(jnp.bfloat16),
                preferred_element_type=jnp.float32)
        + bc1_ref[...], 0.0)
    out_ref[...] = (
        jnp.dot(z.astype(jnp.bfloat16), wc2_ref[...].astype(jnp.bfloat16),
                preferred_element_type=jnp.float32)
        + bc2_ref[...])


def _norm_head(h2, st, g, beta, wc1, bc1, wc2, bc2):
    return pl.pallas_call(
        _norm_head_body,
        grid=(_GRID,),
        in_specs=[
            pl.BlockSpec((_BLK, _D), lambda i: (i, 0)),
            pl.BlockSpec((2, _D), lambda i: (0, 0)),
            pl.BlockSpec((1, _D), lambda i: (0, 0)),
            pl.BlockSpec((1, _D), lambda i: (0, 0)),
            pl.BlockSpec((_D, _D // 2), lambda i: (0, 0)),
            pl.BlockSpec((1, _D // 2), lambda i: (0, 0)),
            pl.BlockSpec((_D // 2, 2), lambda i: (0, 0)),
            pl.BlockSpec((1, 2), lambda i: (0, 0)),
        ],
        out_specs=pl.BlockSpec((_BLK, 2), lambda i: (i, 0)),
        out_shape=jax.ShapeDtypeStruct((_N, 2), jnp.float32),
    )(h2, st, g.reshape(1, _D), beta.reshape(1, _D), wc1,
      bc1.reshape(1, _D // 2), wc2, bc2.reshape(1, 2))


def kernel(x, edge_index, W1a, b1a, W1b, b1b, eps1, g1, beta1,
           W2a, b2a, W2b, b2b, eps2, g2, beta2,
           W3a, b3a, W3b, b3b, eps3, g3, beta3,
           Wc1, bc1, Wc2, bc2):
    src = edge_index[0]
    dst = edge_index[1]
    h = x
    layers = [
        (W1a, b1a, W1b, b1b, eps1, g1, beta1),
        (W2a, b2a, W2b, b2b, eps2, g2, beta2),
        (W3a, b3a, W3b, b3b, eps3, g3, beta3),
    ]
    for li, (wa, ba, wb, bb, eps, g, beta) in enumerate(layers):
        parts = _make_sc_agg()(h, src, dst)
        h2, st = _mm_stats(h, parts, wa, ba, wb, bb, eps)
        if li < 2:
            h = _norm(h2, st, g, beta)
        else:
            out = _norm_head(h2, st, g, beta, Wc1, bc1, Wc2, bc2)
    return out
